# Initial kernel scaffold; baseline (speedup 1.0000x reference)
import jax
import jax.numpy as jnp
from jax.experimental import pallas as pl


def kernel(x, edge_index, layer, node_anchor, W_att, b_att, edge_anchor, W_ew, b_ew, scales, W1, b1, W2, b2):
    raise NotImplementedError("write your pallas kernel here")


# speedup vs baseline: 8.0677x; 8.0677x over previous
import functools

import jax
import jax.numpy as jnp
from jax import lax
from jax.experimental import pallas as pl
from jax.experimental.pallas import tpu as pltpu
from jax.experimental.pallas import tpu_sc as plsc

_NEG = -1e30


def _prep_tc(x, Watt16, batt16, anchor16, Wuv, buv):
    N, D = x.shape
    R = 1000

    def body(x_ref, wa_ref, ba_ref, an_ref, wuv_ref, buv_ref, y_ref, uv_ref):
        xb = x_ref[...]
        s = jnp.dot(xb, wa_ref[...], preferred_element_type=jnp.float32) + ba_ref[...]
        m = jnp.max(s, axis=1, keepdims=True)
        e = jnp.exp(s - m)
        att = e / jnp.sum(e, axis=1, keepdims=True)
        y_ref[...] = xb + jnp.dot(att, an_ref[...], preferred_element_type=jnp.float32)
        uv_ref[...] = jnp.dot(xb, wuv_ref[...], preferred_element_type=jnp.float32) + buv_ref[...]

    full = lambda shape: pl.BlockSpec(shape, lambda i: (0,) * len(shape))
    return pl.pallas_call(
        body,
        grid=(N // R,),
        in_specs=[
            pl.BlockSpec((R, D), lambda i: (i, 0)),
            full((D, 16)), full((1, 16)), full((16, D)),
            full((D, D)), full((1, D)),
        ],
        out_specs=[
            pl.BlockSpec((R, D), lambda i: (i, 0)),
            pl.BlockSpec((R, D), lambda i: (i, 0)),
        ],
        out_shape=[
            jax.ShapeDtypeStruct((N, D), jnp.float32),
            jax.ShapeDtypeStruct((N, D), jnp.float32),
        ],
    )(x, Watt16, batt16, anchor16, Wuv, buv)


def _edge_score_sc(uv, ei0, ei1):
    E = ei0.shape[0]
    N, D = uv.shape
    NW = 32
    per_w = E // NW
    K = 200
    mesh = plsc.VectorSubcoreMesh(core_axis_name="c", subcore_axis_name="s")

    @functools.partial(
        pl.kernel,
        out_type=jax.ShapeDtypeStruct((E, 16), jnp.float32),
        mesh=mesh,
        scratch_types=[
            pltpu.VMEM((K,), jnp.int32),
            pltpu.VMEM((K,), jnp.int32),
            pltpu.VMEM((K, D), jnp.float32),
            pltpu.VMEM((K, D), jnp.float32),
            pltpu.VMEM((K, 16), jnp.float32),
            pltpu.SemaphoreType.DMA,
            pltpu.SemaphoreType.DMA,
        ],
    )
    def k(uv_hbm, e0_hbm, e1_hbm, sc_hbm, i0, i1, rows0, rows1, sbuf, sem0, sem1):
        wid = lax.axis_index("s") * 2 + lax.axis_index("c")
        base = wid * per_w

        def chunk(j, _):
            off = base + j * K
            pltpu.sync_copy(e0_hbm.at[pl.ds(off, K)], i0)
            pltpu.sync_copy(e1_hbm.at[pl.ds(off, K)], i1)
            cp0 = pltpu.async_copy(uv_hbm.at[i0], rows0, sem0)
            cp1 = pltpu.async_copy(uv_hbm.at[i1], rows1, sem1)
            cp0.wait()
            cp1.wait()

            def add(i, _):
                sbuf[i, pl.ds(0, 16)] = rows0[i, pl.ds(0, 16)] + rows1[i, pl.ds(16, 16)]
                return 0
            lax.fori_loop(0, K, add, 0)
            pltpu.sync_copy(sbuf, sc_hbm.at[pl.ds(off, K)])
            return 0
        lax.fori_loop(0, per_w // K, chunk, 0)

    return k(uv, ei0, ei1)


def _edge_prompt_tc(scores, anchor_e16):
    E = scores.shape[0]
    D = anchor_e16.shape[1]
    K = 2000

    def body(s_ref, an_ref, ep_ref):
        s = s_ref[...]
        s = jnp.where(s >= 0, s, 0.01 * s)
        m = jnp.max(s, axis=1, keepdims=True)
        e = jnp.exp(s - m)
        b = e / jnp.sum(e, axis=1, keepdims=True)
        ep_ref[...] = jnp.dot(b, an_ref[...], preferred_element_type=jnp.float32)

    return pl.pallas_call(
        body,
        grid=(E // K,),
        in_specs=[
            pl.BlockSpec((K, 16), lambda i: (i, 0)),
            pl.BlockSpec((16, D), lambda i: (0, 0)),
        ],
        out_specs=pl.BlockSpec((K, D), lambda i: (i, 0)),
        out_shape=jax.ShapeDtypeStruct((E, D), jnp.float32),
    )(scores, anchor_e16)


def _diffuse_sc(ep, y_base, ei0, ei1, cc):
    E = ei0.shape[0]
    N, D = y_base.shape
    NS = 16
    TE = E // NS
    KE = 200
    RC = 40
    NCH = N // RC
    MCH = -(-NCH // NS)
    mesh = plsc.VectorSubcoreMesh(core_axis_name="c", subcore_axis_name="s",
                                  num_cores=1)

    @functools.partial(
        pl.kernel,
        out_type=[jax.ShapeDtypeStruct((4, N, D), jnp.float32),
                  jax.ShapeDtypeStruct((N, D), jnp.float32)],
        mesh=mesh,
        scratch_types=[
            pltpu.VMEM_SHARED((N, D), jnp.float32),
            pltpu.VMEM((KE, D), jnp.float32),
            pltpu.VMEM((KE,), jnp.int32),
            pltpu.VMEM((KE,), jnp.int32),
            pltpu.VMEM((RC, D), jnp.float32),
            pltpu.VMEM((RC, D), jnp.float32),
            pltpu.VMEM((RC, D), jnp.float32),
            pltpu.SemaphoreType.DMA,
        ],
    )
    def k(ep_hbm, yb_hbm, e0_hbm, e1_hbm, ys_hbm, y0_hbm,
          ns, ebuf, i0, i1, ybuf, nbuf, zbuf, sem):
        tid = lax.axis_index("s")
        erow0 = tid * TE
        z16 = jnp.zeros((16,), jnp.float32)
        nv = D // 16

        def zb(i, _):
            zbuf[i // nv, pl.ds((i % nv) * 16, 16)] = z16
            return 0
        lax.fori_loop(0, RC * nv, zb, 0)

        def zero_ns(m, _):
            ci = m * NS + tid

            @pl.when(ci < NCH)
            def _():
                pltpu.sync_copy(zbuf, ns.at[pl.ds(ci * RC, RC)])
            return 0
        lax.fori_loop(0, MCH, zero_ns, 0)
        plsc.subcore_barrier()

        def agg(j, _):
            off = erow0 + j * KE
            pltpu.sync_copy(e0_hbm.at[pl.ds(off, KE)], i0)
            pltpu.sync_copy(e1_hbm.at[pl.ds(off, KE)], i1)
            pltpu.sync_copy(ep_hbm.at[pl.ds(off, KE)], ebuf)
            pltpu.sync_copy(ebuf, ns.at[i0], add=True)
            pltpu.sync_copy(ebuf, ns.at[i1], add=True)
            return 0
        lax.fori_loop(0, TE // KE, agg, 0)
        plsc.subcore_barrier()

        def make_y0(m, _):
            ci = m * NS + tid

            @pl.when(ci < NCH)
            def _():
                r0 = ci * RC
                pltpu.sync_copy(yb_hbm.at[pl.ds(r0, RC)], ybuf)
                pltpu.sync_copy(ns.at[pl.ds(r0, RC)], nbuf)

                def add_ns(i, _):
                    r, q = i // nv, (i % nv) * 16
                    ybuf[r, pl.ds(q, 16)] = ybuf[r, pl.ds(q, 16)] + nbuf[r, pl.ds(q, 16)]
                    return 0
                lax.fori_loop(0, RC * nv, add_ns, 0)
                pltpu.sync_copy(ybuf, y0_hbm.at[pl.ds(r0, RC)])
                pltpu.sync_copy(zbuf, ns.at[pl.ds(r0, RC)])
            return 0
        lax.fori_loop(0, MCH, make_y0, 0)
        plsc.subcore_barrier()

        for t in range(4):
            src = y0_hbm if t == 0 else ys_hbm.at[t - 1]

            def scat(j, _):
                off = erow0 + j * KE
                pltpu.sync_copy(e0_hbm.at[pl.ds(off, KE)], i0)
                pltpu.sync_copy(e1_hbm.at[pl.ds(off, KE)], i1)
                pltpu.async_copy(src.at[i1], ebuf, sem).wait()
                pltpu.sync_copy(ebuf, ns.at[i0], add=True)
                pltpu.async_copy(src.at[i0], ebuf, sem).wait()
                pltpu.sync_copy(ebuf, ns.at[i1], add=True)
                return 0
            lax.fori_loop(0, TE // KE, scat, 0)
            plsc.subcore_barrier()

            def upd(m, _):
                ci = m * NS + tid

                @pl.when(ci < NCH)
                def _():
                    r0 = ci * RC
                    pltpu.sync_copy(src.at[pl.ds(r0, RC)], ybuf)
                    pltpu.sync_copy(ns.at[pl.ds(r0, RC)], nbuf)

                    def step(i, _):
                        r, q = i // nv, (i % nv) * 16
                        ybuf[r, pl.ds(q, 16)] = (0.5 * ybuf[r, pl.ds(q, 16)]
                                                 + cc * nbuf[r, pl.ds(q, 16)])
                        return 0
                    lax.fori_loop(0, RC * nv, step, 0)
                    pltpu.sync_copy(ybuf, ys_hbm.at[t, pl.ds(r0, RC)])
                    pltpu.sync_copy(zbuf, ns.at[pl.ds(r0, RC)])
                return 0
            lax.fori_loop(0, MCH, upd, 0)
            plsc.subcore_barrier()

    return k(ep, y_base, ei0, ei1)


def _mlp_tc(ys, W1s, b1, W2, b2):
    _, N, D = ys.shape
    D2 = W1s.shape[1]
    R = 1000

    def body(ys_ref, w1_ref, b1_ref, w2_ref, b2_ref, o_ref):
        acc = jnp.broadcast_to(b1_ref[...], (R, D2))
        for s in range(4):
            acc = acc + jnp.dot(ys_ref[s], w1_ref[s * D:(s + 1) * D, :],
                                preferred_element_type=jnp.float32)
        h = jnp.maximum(acc, 0.0)
        o_ref[...] = jnp.dot(h, w2_ref[...], preferred_element_type=jnp.float32) + b2_ref[...]

    full = lambda shape: pl.BlockSpec(shape, lambda i: (0,) * len(shape))
    return pl.pallas_call(
        body,
        grid=(N // R,),
        in_specs=[
            pl.BlockSpec((4, R, D), lambda i: (0, i, 0)),
            full(W1s.shape), full((1, D2)), full(W2.shape), full((1, b2.shape[1]))
        ],
        out_specs=pl.BlockSpec((R, b2.shape[1]), lambda i: (i, 0)),
        out_shape=jax.ShapeDtypeStruct((N, b2.shape[1]), jnp.float32),
    )(ys, W1s, b1, W2, b2)


def kernel(x, edge_index, layer, node_anchor, W_att, b_att, edge_anchor,
           W_ew, b_ew, scales, W1, b1, W2, b2):
    N, D = x.shape
    E = edge_index.shape[1]
    A = W_att.shape[1]
    denom = E / N + 1e-06
    cc = 0.5 / denom

    Watt16 = jnp.zeros((D, 16), jnp.float32).at[:, :A].set(W_att)
    batt16 = jnp.full((1, 16), _NEG, jnp.float32).at[0, :A].set(b_att)
    anchor16 = jnp.zeros((16, D), jnp.float32).at[:A].set(node_anchor)
    anchor_e16 = jnp.zeros((16, D), jnp.float32).at[:A].set(edge_anchor)
    Wuv = (jnp.zeros((D, D), jnp.float32)
           .at[:, 0:A].set(W_ew[:D])
           .at[:, 16:16 + A].set(W_ew[D:]))
    buv = (jnp.zeros((1, D), jnp.float32)
           .at[0, 0:A].set(b_ew)
           .at[0, A:16].set(_NEG))
    W1s = W1 * jnp.repeat(scales, D)[:, None]

    ei0 = edge_index[0]
    ei1 = edge_index[1]

    y_base, uv = _prep_tc(x, Watt16, batt16, anchor16, Wuv, buv)
    scores = _edge_score_sc(uv, ei0, ei1)
    ep = _edge_prompt_tc(scores, anchor_e16)
    ys, _ = _diffuse_sc(ep, y_base, ei0, ei1, cc)
    final_x = _mlp_tc(ys, W1s, b1.reshape(1, -1), W2, b2.reshape(1, -1))
    return (final_x, ep)


# speedup vs baseline: 9.7440x; 1.2078x over previous
import functools

import jax
import jax.numpy as jnp
from jax import lax
from jax.experimental import pallas as pl
from jax.experimental.pallas import tpu as pltpu
from jax.experimental.pallas import tpu_sc as plsc

_NEG = -1e30


def _prep_tc(x, Watt16, batt16, anchor16, Wuv, buv):
    N, D = x.shape
    R = 1000

    def body(x_ref, wa_ref, ba_ref, an_ref, wuv_ref, buv_ref, y_ref, uv_ref):
        xb = x_ref[...]
        s = jnp.dot(xb, wa_ref[...], preferred_element_type=jnp.float32) + ba_ref[...]
        m = jnp.max(s, axis=1, keepdims=True)
        e = jnp.exp(s - m)
        att = e / jnp.sum(e, axis=1, keepdims=True)
        y_ref[...] = xb + jnp.dot(att, an_ref[...], preferred_element_type=jnp.float32)
        uv_ref[...] = jnp.dot(xb, wuv_ref[...], preferred_element_type=jnp.float32) + buv_ref[...]

    full = lambda shape: pl.BlockSpec(shape, lambda i: (0,) * len(shape))
    return pl.pallas_call(
        body,
        grid=(N // R,),
        in_specs=[
            pl.BlockSpec((R, D), lambda i: (i, 0)),
            full((D, 16)), full((1, 16)), full((16, D)),
            full((D, D)), full((1, D)),
        ],
        out_specs=[
            pl.BlockSpec((R, D), lambda i: (i, 0)),
            pl.BlockSpec((R, D), lambda i: (i, 0)),
        ],
        out_shape=[
            jax.ShapeDtypeStruct((N, D), jnp.float32),
            jax.ShapeDtypeStruct((N, D), jnp.float32),
        ],
    )(x, Watt16, batt16, anchor16, Wuv, buv)


def _edge_score_sc(uv, ei0, ei1):
    E = ei0.shape[0]
    N, D = uv.shape
    NW = 32
    per_w = E // NW
    K = 200
    mesh = plsc.VectorSubcoreMesh(core_axis_name="c", subcore_axis_name="s")

    @functools.partial(
        pl.kernel,
        out_type=jax.ShapeDtypeStruct((E, 16), jnp.float32),
        mesh=mesh,
        scratch_types=[
            pltpu.VMEM((K,), jnp.int32),
            pltpu.VMEM((K,), jnp.int32),
            pltpu.VMEM((K, D), jnp.float32),
            pltpu.VMEM((K, D), jnp.float32),
            pltpu.VMEM((K, 16), jnp.float32),
            pltpu.SemaphoreType.DMA,
            pltpu.SemaphoreType.DMA,
        ],
    )
    def k(uv_hbm, e0_hbm, e1_hbm, sc_hbm, i0, i1, rows0, rows1, sbuf, sem0, sem1):
        wid = lax.axis_index("s") * 2 + lax.axis_index("c")
        base = wid * per_w

        def chunk(j, _):
            off = base + j * K
            pltpu.sync_copy(e0_hbm.at[pl.ds(off, K)], i0)
            pltpu.sync_copy(e1_hbm.at[pl.ds(off, K)], i1)
            cp0 = pltpu.async_copy(uv_hbm.at[i0], rows0, sem0)
            cp1 = pltpu.async_copy(uv_hbm.at[i1], rows1, sem1)
            cp0.wait()
            cp1.wait()

            def add(i, _):
                sbuf[i, pl.ds(0, 16)] = rows0[i, pl.ds(0, 16)] + rows1[i, pl.ds(16, 16)]
                return 0
            lax.fori_loop(0, K, add, 0)
            pltpu.sync_copy(sbuf, sc_hbm.at[pl.ds(off, K)])
            return 0
        lax.fori_loop(0, per_w // K, chunk, 0)

    return k(uv, ei0, ei1)


def _edge_prompt_tc(scores, anchor_e16):
    E = scores.shape[0]
    D = anchor_e16.shape[1]
    K = 2000

    def body(s_ref, an_ref, ep_ref):
        s = s_ref[...]
        s = jnp.where(s >= 0, s, 0.01 * s)
        m = jnp.max(s, axis=1, keepdims=True)
        e = jnp.exp(s - m)
        b = e / jnp.sum(e, axis=1, keepdims=True)
        ep_ref[...] = jnp.dot(b, an_ref[...], preferred_element_type=jnp.float32)

    return pl.pallas_call(
        body,
        grid=(E // K,),
        in_specs=[
            pl.BlockSpec((K, 16), lambda i: (i, 0)),
            pl.BlockSpec((16, D), lambda i: (0, 0)),
        ],
        out_specs=pl.BlockSpec((K, D), lambda i: (i, 0)),
        out_shape=jax.ShapeDtypeStruct((E, D), jnp.float32),
    )(scores, anchor_e16)


def _diffuse_sc(ep, y_base, ei0, ei1, cc):
    E = ei0.shape[0]
    N, D = y_base.shape
    NS = 16
    KE = 128
    NCHE = E // KE
    RC = 80
    NCH = N // RC
    MCH = -(-NCH // NS)
    nv = D // 16
    BODIES = -(-(-(-NCHE // NS)) // 2)
    mesh = plsc.VectorSubcoreMesh(core_axis_name="c", subcore_axis_name="s",
                                  num_cores=1)

    @functools.partial(
        pl.kernel,
        out_type=[jax.ShapeDtypeStruct((4, N, D), jnp.float32),
                  jax.ShapeDtypeStruct((N, D), jnp.float32)],
        mesh=mesh,
        scratch_types=[
            pltpu.VMEM_SHARED((N, D), jnp.float32),
            pltpu.VMEM((KE, D), jnp.float32),
            pltpu.VMEM((KE, D), jnp.float32),
            pltpu.VMEM((8, KE), jnp.int32),
            pltpu.VMEM((8, KE), jnp.int32),
            pltpu.VMEM((8, KE), jnp.int32),
            pltpu.VMEM((8, KE), jnp.int32),
            pltpu.VMEM((RC, D), jnp.float32),
            pltpu.SemaphoreType.DMA,
            pltpu.SemaphoreType.DMA,
            pltpu.SemaphoreType.DMA,
            pltpu.SemaphoreType.DMA,
        ],
    )
    def k(ep_hbm, yb_hbm, e0_hbm, e1_hbm, ys_hbm, y0_hbm,
          ns, buf0, buf1, i0A, i1A, i0B, i1B, zbuf, sgA, sgB, siA, siB):
        tid = lax.axis_index("s")
        xtra = NCHE - (NCHE // NS) * NS
        base = (NCHE // NS) * tid + jnp.minimum(tid, xtra)
        cnt = jnp.where(tid < xtra, NCHE // NS + 1, NCHE // NS)
        z16 = jnp.zeros((16,), jnp.float32)

        def idx_issue(ci, i0b, i1b, si):
            pltpu.async_copy(e0_hbm.at[pl.ds(ci * KE, KE)], i0b.at[0], si)
            pltpu.async_copy(e1_hbm.at[pl.ds(ci * KE, KE)], i1b.at[0], si)

        def idx_wait(ci, i0b, i1b, si):
            pltpu.make_async_copy(e0_hbm.at[pl.ds(ci * KE, KE)], i0b.at[0], si).wait()
            pltpu.make_async_copy(e1_hbm.at[pl.ds(ci * KE, KE)], i1b.at[0], si).wait()

        def zb(i, _):
            zbuf[i // nv, pl.ds((i % nv) * 16, 16)] = z16
            return 0
        lax.fori_loop(0, RC * nv, zb, 0)

        def zero_ns(m, _):
            ci = m * NS + tid

            @pl.when(ci < NCH)
            def _():
                pltpu.sync_copy(zbuf, ns.at[pl.ds(ci * RC, RC)])
            return 0
        lax.fori_loop(0, MCH, zero_ns, 0)
        plsc.subcore_barrier()

        def scatter_phase(src):
            idx_issue(base, i0A, i1A, siA)
            idx_issue(base + 1, i0B, i1B, siB)

            def body(m, _):
                cA = base + 2 * m
                cB = cA + 1
                okA = 2 * m < cnt
                okB = 2 * m + 1 < cnt

                @pl.when(okA)
                def _():
                    idx_wait(cA, i0A, i1A, siA)
                    if src is None:
                        pltpu.async_copy(ep_hbm.at[pl.ds(cA * KE, KE)], buf0, sgA)
                    else:
                        pltpu.async_copy(src.at[i1A.at[0]], buf0, sgA)

                @pl.when(okB)
                def _():
                    idx_wait(cB, i0B, i1B, siB)
                    if src is None:
                        pltpu.async_copy(ep_hbm.at[pl.ds(cB * KE, KE)], buf1, sgB)
                    else:
                        pltpu.async_copy(src.at[i1B.at[0]], buf1, sgB)

                @pl.when(okA)
                def _():
                    if src is None:
                        pltpu.make_async_copy(ep_hbm.at[pl.ds(cA * KE, KE)], buf0, sgA).wait()
                        pltpu.sync_copy(buf0, ns.at[i0A.at[0]], add=True)
                        pltpu.sync_copy(buf0, ns.at[i1A.at[0]], add=True)
                    else:
                        pltpu.make_async_copy(src.at[i1A.at[0]], buf0, sgA).wait()
                        pltpu.sync_copy(buf0, ns.at[i0A.at[0]], add=True)
                        pltpu.async_copy(src.at[i0A.at[0]], buf0, sgA)

                @pl.when(okB)
                def _():
                    if src is None:
                        pltpu.make_async_copy(ep_hbm.at[pl.ds(cB * KE, KE)], buf1, sgB).wait()
                        pltpu.sync_copy(buf1, ns.at[i0B.at[0]], add=True)
                        pltpu.sync_copy(buf1, ns.at[i1B.at[0]], add=True)
                    else:
                        pltpu.make_async_copy(src.at[i1B.at[0]], buf1, sgB).wait()
                        pltpu.sync_copy(buf1, ns.at[i0B.at[0]], add=True)
                        pltpu.async_copy(src.at[i0B.at[0]], buf1, sgB)

                if src is not None:
                    @pl.when(okA)
                    def _():
                        pltpu.make_async_copy(src.at[i0A.at[0]], buf0, sgA).wait()
                        pltpu.sync_copy(buf0, ns.at[i1A.at[0]], add=True)

                    @pl.when(okB)
                    def _():
                        pltpu.make_async_copy(src.at[i0B.at[0]], buf1, sgB).wait()
                        pltpu.sync_copy(buf1, ns.at[i1B.at[0]], add=True)

                @pl.when(2 * m + 2 < cnt)
                def _():
                    idx_issue(cA + 2, i0A, i1A, siA)

                @pl.when(2 * m + 3 < cnt)
                def _():
                    idx_issue(cB + 2, i0B, i1B, siB)
                return 0
            lax.fori_loop(0, BODIES, body, 0)
            plsc.subcore_barrier()

        def update_phase(src, dst, first):
            def upd(m, _):
                ci = m * NS + tid

                @pl.when(ci < NCH)
                def _():
                    r0 = ci * RC
                    cpy = pltpu.async_copy(src.at[pl.ds(r0, RC)], buf0.at[pl.ds(0, RC)], sgA)
                    cpn = pltpu.async_copy(ns.at[pl.ds(r0, RC)], buf1.at[pl.ds(0, RC)], sgB)
                    cpy.wait()
                    cpn.wait()

                    def f(i, _):
                        r, q = i // nv, (i % nv) * 16
                        if first:
                            buf0[r, pl.ds(q, 16)] = buf0[r, pl.ds(q, 16)] + buf1[r, pl.ds(q, 16)]
                        else:
                            buf0[r, pl.ds(q, 16)] = (0.5 * buf0[r, pl.ds(q, 16)]
                                                     + cc * buf1[r, pl.ds(q, 16)])
                        return 0
                    lax.fori_loop(0, RC * nv, f, 0)
                    pltpu.sync_copy(buf0.at[pl.ds(0, RC)], dst.at[pl.ds(r0, RC)])
                    pltpu.sync_copy(zbuf, ns.at[pl.ds(r0, RC)])
                return 0
            lax.fori_loop(0, MCH, upd, 0)
            plsc.subcore_barrier()

        scatter_phase(None)
        update_phase(yb_hbm, y0_hbm, True)

        for t in range(4):
            src = y0_hbm if t == 0 else ys_hbm.at[t - 1]
            scatter_phase(src)
            update_phase(src, ys_hbm.at[t], False)

    return k(ep, y_base, ei0, ei1)


def _mlp_tc(ys, W1s, b1, W2, b2):
    _, N, D = ys.shape
    D2 = W1s.shape[1]
    R = 1000

    def body(ys_ref, w1_ref, b1_ref, w2_ref, b2_ref, o_ref):
        acc = jnp.broadcast_to(b1_ref[...], (R, D2))
        for s in range(4):
            acc = acc + jnp.dot(ys_ref[s], w1_ref[s * D:(s + 1) * D, :],
                                preferred_element_type=jnp.float32)
        h = jnp.maximum(acc, 0.0)
        o_ref[...] = jnp.dot(h, w2_ref[...], preferred_element_type=jnp.float32) + b2_ref[...]

    full = lambda shape: pl.BlockSpec(shape, lambda i: (0,) * len(shape))
    return pl.pallas_call(
        body,
        grid=(N // R,),
        in_specs=[
            pl.BlockSpec((4, R, D), lambda i: (0, i, 0)),
            full(W1s.shape), full((1, D2)), full(W2.shape), full((1, b2.shape[1]))
        ],
        out_specs=pl.BlockSpec((R, b2.shape[1]), lambda i: (i, 0)),
        out_shape=jax.ShapeDtypeStruct((N, b2.shape[1]), jnp.float32),
    )(ys, W1s, b1, W2, b2)


def kernel(x, edge_index, layer, node_anchor, W_att, b_att, edge_anchor,
           W_ew, b_ew, scales, W1, b1, W2, b2):
    N, D = x.shape
    E = edge_index.shape[1]
    A = W_att.shape[1]
    denom = E / N + 1e-06
    cc = 0.5 / denom

    Watt16 = jnp.zeros((D, 16), jnp.float32).at[:, :A].set(W_att)
    batt16 = jnp.full((1, 16), _NEG, jnp.float32).at[0, :A].set(b_att)
    anchor16 = jnp.zeros((16, D), jnp.float32).at[:A].set(node_anchor)
    anchor_e16 = jnp.zeros((16, D), jnp.float32).at[:A].set(edge_anchor)
    Wuv = (jnp.zeros((D, D), jnp.float32)
           .at[:, 0:A].set(W_ew[:D])
           .at[:, 16:16 + A].set(W_ew[D:]))
    buv = (jnp.zeros((1, D), jnp.float32)
           .at[0, 0:A].set(b_ew)
           .at[0, A:16].set(_NEG))
    W1s = W1 * jnp.repeat(scales, D)[:, None]

    ei0 = edge_index[0]
    ei1 = edge_index[1]

    y_base, uv = _prep_tc(x, Watt16, batt16, anchor16, Wuv, buv)
    scores = _edge_score_sc(uv, ei0, ei1)
    ep = _edge_prompt_tc(scores, anchor_e16)
    ys, _ = _diffuse_sc(ep, y_base, ei0, ei1, cc)
    final_x = _mlp_tc(ys, W1s, b1.reshape(1, -1), W2, b2.reshape(1, -1))
    return (final_x, ep)


# speedup vs baseline: 13.0679x; 1.3411x over previous
import functools

import jax
import jax.numpy as jnp
from jax import lax
from jax.experimental import pallas as pl
from jax.experimental.pallas import tpu as pltpu
from jax.experimental.pallas import tpu_sc as plsc

_NEG = -1e30


def _prep_tc(x, Watt16, batt16, anchor16, Wuv, buv):
    N, D = x.shape
    R = 1000

    def body(x_ref, wa_ref, ba_ref, an_ref, wuv_ref, buv_ref, y_ref, uv_ref,
             fl_ref):
        xb = x_ref[...]
        s = jnp.dot(xb, wa_ref[...], preferred_element_type=jnp.float32) + ba_ref[...]
        m = jnp.max(s, axis=1, keepdims=True)
        e = jnp.exp(s - m)
        att = e / jnp.sum(e, axis=1, keepdims=True)
        y_ref[...] = xb + jnp.dot(att, an_ref[...], preferred_element_type=jnp.float32)
        uv_ref[...] = jnp.dot(xb, wuv_ref[...], preferred_element_type=jnp.float32) + buv_ref[...]
        fl_ref[...] = jnp.zeros((2, 16), jnp.int32)

    full = lambda shape: pl.BlockSpec(shape, lambda i: (0,) * len(shape))
    return pl.pallas_call(
        body,
        grid=(N // R,),
        in_specs=[
            pl.BlockSpec((R, D), lambda i: (i, 0)),
            full((D, 16)), full((1, 16)), full((16, D)),
            full((D, D)), full((1, D)),
        ],
        out_specs=[
            pl.BlockSpec((R, D), lambda i: (i, 0)),
            pl.BlockSpec((R, D), lambda i: (i, 0)),
            full((2, 16)),
        ],
        out_shape=[
            jax.ShapeDtypeStruct((N, D), jnp.float32),
            jax.ShapeDtypeStruct((N, D), jnp.float32),
            jax.ShapeDtypeStruct((2, 16), jnp.int32),
        ],
    )(x, Watt16, batt16, anchor16, Wuv, buv)


def _edge_score_sc(uv, ei0, ei1):
    E = ei0.shape[0]
    N, D = uv.shape
    NW = 32
    per_w = E // NW
    K = 200
    mesh = plsc.VectorSubcoreMesh(core_axis_name="c", subcore_axis_name="s")

    @functools.partial(
        pl.kernel,
        out_type=jax.ShapeDtypeStruct((E, 16), jnp.float32),
        mesh=mesh,
        scratch_types=[
            pltpu.VMEM((K,), jnp.int32),
            pltpu.VMEM((K,), jnp.int32),
            pltpu.VMEM((K, D), jnp.float32),
            pltpu.VMEM((K, D), jnp.float32),
            pltpu.VMEM((K, 16), jnp.float32),
            pltpu.SemaphoreType.DMA,
            pltpu.SemaphoreType.DMA,
        ],
    )
    def k(uv_hbm, e0_hbm, e1_hbm, sc_hbm, i0, i1, rows0, rows1, sbuf, sem0, sem1):
        wid = lax.axis_index("s") * 2 + lax.axis_index("c")
        base = wid * per_w

        def chunk(j, _):
            off = base + j * K
            pltpu.sync_copy(e0_hbm.at[pl.ds(off, K)], i0)
            pltpu.sync_copy(e1_hbm.at[pl.ds(off, K)], i1)
            cp0 = pltpu.async_copy(uv_hbm.at[i0], rows0, sem0)
            cp1 = pltpu.async_copy(uv_hbm.at[i1], rows1, sem1)
            cp0.wait()
            cp1.wait()

            def add(i, _):
                sbuf[i, pl.ds(0, 16)] = rows0[i, pl.ds(0, 16)] + rows1[i, pl.ds(16, 16)]
                return 0
            lax.fori_loop(0, K, add, 0)
            pltpu.sync_copy(sbuf, sc_hbm.at[pl.ds(off, K)])
            return 0
        lax.fori_loop(0, per_w // K, chunk, 0)

    return k(uv, ei0, ei1)


def _edge_prompt_tc(scores, anchor_e16):
    E = scores.shape[0]
    D = anchor_e16.shape[1]
    K = 2000

    def body(s_ref, an_ref, ep_ref):
        s = s_ref[...]
        s = jnp.where(s >= 0, s, 0.01 * s)
        m = jnp.max(s, axis=1, keepdims=True)
        e = jnp.exp(s - m)
        b = e / jnp.sum(e, axis=1, keepdims=True)
        ep_ref[...] = jnp.dot(b, an_ref[...], preferred_element_type=jnp.float32)

    return pl.pallas_call(
        body,
        grid=(E // K,),
        in_specs=[
            pl.BlockSpec((K, 16), lambda i: (i, 0)),
            pl.BlockSpec((16, D), lambda i: (0, 0)),
        ],
        out_specs=pl.BlockSpec((K, D), lambda i: (i, 0)),
        out_shape=jax.ShapeDtypeStruct((E, D), jnp.float32),
    )(scores, anchor_e16)


def _diffuse_sc(ep, y_base, ei0, ei1, flags0, cc):
    E = ei0.shape[0]
    N, D = y_base.shape
    NS = 16
    KE = 64
    NCHE = E // KE
    HCH = NCHE // 2
    RC = 40
    NCH = N // RC
    MEXP = -(-NCH // NS)
    MB32 = -(-NCH // 32)
    nv = D // 16
    BODIES = -(-(-(-HCH // NS)) // 2)
    mesh = plsc.VectorSubcoreMesh(core_axis_name="c", subcore_axis_name="s")

    @functools.partial(
        pl.kernel,
        out_type=[jax.ShapeDtypeStruct((4, N, D), jnp.float32),
                  jax.ShapeDtypeStruct((N, D), jnp.float32),
                  jax.ShapeDtypeStruct((2, N, D), jnp.float32)],
        mesh=mesh,
        scratch_types=[
            pltpu.VMEM_SHARED((N, D), jnp.float32),
            pltpu.VMEM((KE, D), jnp.float32),
            pltpu.VMEM((KE, D), jnp.float32),
            pltpu.VMEM((KE, D), jnp.float32),
            pltpu.VMEM((KE, D), jnp.float32),
            pltpu.VMEM((8, KE), jnp.int32),
            pltpu.VMEM((8, KE), jnp.int32),
            pltpu.VMEM((8, KE), jnp.int32),
            pltpu.VMEM((8, KE), jnp.int32),
            pltpu.VMEM((8, KE), jnp.int32),
            pltpu.VMEM((8, KE), jnp.int32),
            pltpu.VMEM((8, KE), jnp.int32),
            pltpu.VMEM((8, KE), jnp.int32),
            pltpu.VMEM((40, D), jnp.float32),
            pltpu.VMEM((16,), jnp.int32),
            pltpu.VMEM((16,), jnp.int32),
            pltpu.SemaphoreType.DMA,
            pltpu.SemaphoreType.DMA,
            pltpu.SemaphoreType.DMA,
            pltpu.SemaphoreType.DMA,
            pltpu.SemaphoreType.DMA,
            pltpu.SemaphoreType.DMA,
            pltpu.SemaphoreType.DMA,
            pltpu.SemaphoreType.DMA,
            pltpu.SemaphoreType.DMA,
            pltpu.SemaphoreType.DMA,
        ],
    )
    def k(ep_hbm, yb_hbm, e0_hbm, e1_hbm, flags, ys_hbm, y0_hbm, pns_hbm,
          ns, buf0, buf1, buf2, buf3, i0A, i1A, i0B, i1B, jA, jB, j2A, j2B,
          zbuf, fbuf, gbuf,
          sgA, sgB, sgA2, sgB2, siA, siB, saA, saB, saA2, saB2):
        c = lax.axis_index("c")
        tid = lax.axis_index("s")
        w32 = tid * 2 + c
        xtra = HCH - (HCH // NS) * NS
        base = c * HCH + (HCH // NS) * tid + jnp.minimum(tid, xtra)
        cnt = jnp.where(tid < xtra, HCH // NS + 1, HCH // NS)
        z16 = jnp.zeros((16,), jnp.float32)

        def idx_issue(ci, i0b, i1b, si):
            pltpu.async_copy(e0_hbm.at[pl.ds(ci * KE, KE)], i0b.at[0], si)
            pltpu.async_copy(e1_hbm.at[pl.ds(ci * KE, KE)], i1b.at[0], si)

        def idx_wait(ci, i0b, i1b, si):
            pltpu.make_async_copy(e0_hbm.at[pl.ds(ci * KE, KE)], i0b.at[0], si).wait()
            pltpu.make_async_copy(e1_hbm.at[pl.ds(ci * KE, KE)], i1b.at[0], si).wait()

        def zb(i, _):
            zbuf[i // nv, pl.ds((i % nv) * 16, 16)] = z16
            return 0
        lax.fori_loop(0, 40 * nv, zb, 0)

        @pl.when(tid == 0)
        def _():
            fbuf[...] = jnp.zeros((16,), jnp.int32)
            pltpu.sync_copy(fbuf, flags.at[c])

        def zero_ns(m, _):
            ci = m * NS + tid

            @pl.when(ci < NCH)
            def _():
                pltpu.sync_copy(zbuf, ns.at[pl.ds(ci * RC, RC)])
            return 0
        lax.fori_loop(0, MEXP, zero_ns, 0)
        plsc.subcore_barrier()

        bno = [0]

        def cross_barrier():
            bno[0] += 1
            B = bno[0]
            plsc.subcore_barrier()

            @pl.when(tid == 0)
            def _():
                fbuf[...] = jnp.full((16,), B, jnp.int32)
                pltpu.sync_copy(fbuf, flags.at[c])

            def bd(i, v):
                @pl.when(v < B)
                def _():
                    pltpu.sync_copy(flags.at[1 - c], gbuf)
                return jnp.where(v < B, gbuf[...][0], v)
            lax.fori_loop(0, 256, bd, jnp.int32(-1))
            plsc.subcore_barrier()

        def jcopy(idst, isrc):
            for q in range(KE // 16):
                idst[0, pl.ds(q * 16, 16)] = isrc[0, pl.ds(q * 16, 16)]

        def scatter_phase(src):
            idx_issue(base, i0A, i1A, siA)
            idx_issue(base + 1, i0B, i1B, siB)
            a2src = buf0 if src is None else buf2
            b2src = buf1 if src is None else buf3

            def wait_addsA():
                pltpu.make_async_copy(buf0, ns.at[jA.at[0]], saA).wait()
                pltpu.make_async_copy(a2src, ns.at[j2A.at[0]], saA2).wait()

            def wait_addsB():
                pltpu.make_async_copy(buf1, ns.at[jB.at[0]], saB).wait()
                pltpu.make_async_copy(b2src, ns.at[j2B.at[0]], saB2).wait()

            def body(m, _):
                cA = base + 2 * m
                cB = cA + 1
                okA = 2 * m < cnt
                okB = 2 * m + 1 < cnt

                @pl.when((m >= 1) & (2 * m - 2 < cnt))
                def _():
                    wait_addsA()

                @pl.when((m >= 1) & (2 * m - 1 < cnt))
                def _():
                    wait_addsB()

                @pl.when(okA)
                def _():
                    idx_wait(cA, i0A, i1A, siA)
                    if src is None:
                        pltpu.async_copy(ep_hbm.at[pl.ds(cA * KE, KE)], buf0, sgA)
                    else:
                        pltpu.async_copy(src.at[i1A.at[0]], buf0, sgA)
                        pltpu.async_copy(src.at[i0A.at[0]], buf2, sgA2)

                @pl.when(okB)
                def _():
                    idx_wait(cB, i0B, i1B, siB)
                    if src is None:
                        pltpu.async_copy(ep_hbm.at[pl.ds(cB * KE, KE)], buf1, sgB)
                    else:
                        pltpu.async_copy(src.at[i1B.at[0]], buf1, sgB)
                        pltpu.async_copy(src.at[i0B.at[0]], buf3, sgB2)

                @pl.when(okA)
                def _():
                    jcopy(jA, i0A)
                    jcopy(j2A, i1A)
                    if src is None:
                        pltpu.make_async_copy(ep_hbm.at[pl.ds(cA * KE, KE)], buf0, sgA).wait()
                        pltpu.async_copy(buf0, ns.at[jA.at[0]], saA, add=True)
                        pltpu.async_copy(buf0, ns.at[j2A.at[0]], saA2, add=True)
                    else:
                        pltpu.make_async_copy(src.at[i1A.at[0]], buf0, sgA).wait()
                        pltpu.async_copy(buf0, ns.at[jA.at[0]], saA, add=True)
                        pltpu.make_async_copy(src.at[i0A.at[0]], buf2, sgA2).wait()
                        pltpu.async_copy(buf2, ns.at[j2A.at[0]], saA2, add=True)

                @pl.when(okB)
                def _():
                    jcopy(jB, i0B)
                    jcopy(j2B, i1B)
                    if src is None:
                        pltpu.make_async_copy(ep_hbm.at[pl.ds(cB * KE, KE)], buf1, sgB).wait()
                        pltpu.async_copy(buf1, ns.at[jB.at[0]], saB, add=True)
                        pltpu.async_copy(buf1, ns.at[j2B.at[0]], saB2, add=True)
                    else:
                        pltpu.make_async_copy(src.at[i1B.at[0]], buf1, sgB).wait()
                        pltpu.async_copy(buf1, ns.at[jB.at[0]], saB, add=True)
                        pltpu.make_async_copy(src.at[i0B.at[0]], buf3, sgB2).wait()
                        pltpu.async_copy(buf3, ns.at[j2B.at[0]], saB2, add=True)

                @pl.when(2 * m + 2 < cnt)
                def _():
                    idx_issue(cA + 2, i0A, i1A, siA)

                @pl.when(2 * m + 3 < cnt)
                def _():
                    idx_issue(cB + 2, i0B, i1B, siB)
                return 0
            lax.fori_loop(0, BODIES, body, 0)

            @pl.when(2 * (BODIES - 1) < cnt)
            def _():
                wait_addsA()

            @pl.when(2 * BODIES - 1 < cnt)
            def _():
                wait_addsB()
            plsc.subcore_barrier()

        def export_phase():
            def ex(m, _):
                ci = m * NS + tid

                @pl.when(ci < NCH)
                def _():
                    r0 = ci * RC
                    pltpu.sync_copy(ns.at[pl.ds(r0, RC)], pns_hbm.at[c, pl.ds(r0, RC)])
                    pltpu.sync_copy(zbuf, ns.at[pl.ds(r0, RC)])
                return 0
            lax.fori_loop(0, MEXP, ex, 0)

        def build_phase(src, dst, first):
            def bd(m, _):
                ci = m * 32 + w32

                @pl.when(ci < NCH)
                def _():
                    r0 = ci * RC
                    cpy = pltpu.async_copy(src.at[pl.ds(r0, RC)], buf0.at[pl.ds(0, RC)], sgA)
                    cpa = pltpu.async_copy(pns_hbm.at[0, pl.ds(r0, RC)], buf1.at[pl.ds(0, RC)], sgB)
                    cpy.wait()
                    cpa.wait()

                    def f1(i, _):
                        r, q = i // nv, (i % nv) * 16
                        if first:
                            buf0[r, pl.ds(q, 16)] = buf0[r, pl.ds(q, 16)] + buf1[r, pl.ds(q, 16)]
                        else:
                            buf0[r, pl.ds(q, 16)] = (0.5 * buf0[r, pl.ds(q, 16)]
                                                     + cc * buf1[r, pl.ds(q, 16)])
                        return 0
                    lax.fori_loop(0, RC * nv, f1, 0)
                    pltpu.sync_copy(pns_hbm.at[1, pl.ds(r0, RC)], buf1.at[pl.ds(0, RC)])

                    def f2(i, _):
                        r, q = i // nv, (i % nv) * 16
                        if first:
                            buf0[r, pl.ds(q, 16)] = buf0[r, pl.ds(q, 16)] + buf1[r, pl.ds(q, 16)]
                        else:
                            buf0[r, pl.ds(q, 16)] = buf0[r, pl.ds(q, 16)] + cc * buf1[r, pl.ds(q, 16)]
                        return 0
                    lax.fori_loop(0, RC * nv, f2, 0)
                    pltpu.sync_copy(buf0.at[pl.ds(0, RC)], dst.at[pl.ds(r0, RC)])
                return 0
            lax.fori_loop(0, MB32, bd, 0)

        scatter_phase(None)
        export_phase()
        cross_barrier()
        build_phase(yb_hbm, y0_hbm, True)
        cross_barrier()

        for t in range(4):
            src = y0_hbm if t == 0 else ys_hbm.at[t - 1]
            scatter_phase(src)
            export_phase()
            cross_barrier()
            build_phase(src, ys_hbm.at[t], False)
            if t < 3:
                cross_barrier()

    return k(ep, y_base, ei0, ei1, flags0)


def _mlp_tc(ys, W1s, b1, W2, b2):
    _, N, D = ys.shape
    D2 = W1s.shape[1]
    R = 1000

    def body(ys_ref, w1_ref, b1_ref, w2_ref, b2_ref, o_ref):
        acc = jnp.broadcast_to(b1_ref[...], (R, D2))
        for s in range(4):
            acc = acc + jnp.dot(ys_ref[s], w1_ref[s * D:(s + 1) * D, :],
                                preferred_element_type=jnp.float32)
        h = jnp.maximum(acc, 0.0)
        o_ref[...] = jnp.dot(h, w2_ref[...], preferred_element_type=jnp.float32) + b2_ref[...]

    full = lambda shape: pl.BlockSpec(shape, lambda i: (0,) * len(shape))
    return pl.pallas_call(
        body,
        grid=(N // R,),
        in_specs=[
            pl.BlockSpec((4, R, D), lambda i: (0, i, 0)),
            full(W1s.shape), full((1, D2)), full(W2.shape), full((1, b2.shape[1]))
        ],
        out_specs=pl.BlockSpec((R, b2.shape[1]), lambda i: (i, 0)),
        out_shape=jax.ShapeDtypeStruct((N, b2.shape[1]), jnp.float32),
    )(ys, W1s, b1, W2, b2)


def kernel(x, edge_index, layer, node_anchor, W_att, b_att, edge_anchor,
           W_ew, b_ew, scales, W1, b1, W2, b2):
    N, D = x.shape
    E = edge_index.shape[1]
    A = W_att.shape[1]
    denom = E / N + 1e-06
    cc = 0.5 / denom

    Watt16 = jnp.zeros((D, 16), jnp.float32).at[:, :A].set(W_att)
    batt16 = jnp.full((1, 16), _NEG, jnp.float32).at[0, :A].set(b_att)
    anchor16 = jnp.zeros((16, D), jnp.float32).at[:A].set(node_anchor)
    anchor_e16 = jnp.zeros((16, D), jnp.float32).at[:A].set(edge_anchor)
    Wuv = (jnp.zeros((D, D), jnp.float32)
           .at[:, 0:A].set(W_ew[:D])
           .at[:, 16:16 + A].set(W_ew[D:]))
    buv = (jnp.zeros((1, D), jnp.float32)
           .at[0, 0:A].set(b_ew)
           .at[0, A:16].set(_NEG))
    W1s = W1 * jnp.repeat(scales, D)[:, None]

    ei0 = edge_index[0]
    ei1 = edge_index[1]

    y_base, uv, flags0 = _prep_tc(x, Watt16, batt16, anchor16, Wuv, buv)
    scores = _edge_score_sc(uv, ei0, ei1)
    ep = _edge_prompt_tc(scores, anchor_e16)
    ys, _, _ = _diffuse_sc(ep, y_base, ei0, ei1, flags0, cc)
    final_x = _mlp_tc(ys, W1s, b1.reshape(1, -1), W2, b2.reshape(1, -1))
    return (final_x, ep)
